# baseline (speedup 1.0000x reference)
import functools

import jax
import jax.numpy as jnp
from jax import lax
from jax.experimental import pallas as pl
from jax.experimental.pallas import tpu as pltpu


def _body(cb_ref, x_ref, rot_ref, o_ref, *, n_pos, scale):
    rot = rot_ref[...]
    xb = x_ref[...].reshape(-1, rot.shape[0])
    y = lax.dot_general(xb, rot, (((1,), (1,)), ((), ())),
                        preferred_element_type=jnp.float32)
    a = jnp.abs(y)
    c = [cb_ref[0, n_pos + j] * scale for j in range(n_pos)]
    mids = [(c[j - 1] + c[j]) * 0.5 for j in range(1, n_pos)]

    def tree(lo, hi):
        if lo == hi:
            return jnp.full_like(a, c[lo])
        m = (lo + hi) // 2
        return jnp.where(a > mids[m], tree(m + 1, hi), tree(lo, m))

    q = tree(0, n_pos - 1)
    yq = jnp.where(y < 0.0, -q, q)
    o_ref[...] = jnp.dot(yq, rot,
                         preferred_element_type=jnp.float32).reshape(x_ref.shape)


def kernel(x, rotation, codebook):
    b, s, dim = x.shape
    scale = 1.0 / (dim ** 0.5)
    k = codebook.shape[0]
    n_pos = k // 2
    cb2 = codebook.reshape(1, k)

    x2 = x.reshape(b, s // 2, 2 * dim)
    rot2 = jnp.kron(jnp.eye(2, dtype=rotation.dtype), rotation)

    blk = 4
    while b % blk:
        blk //= 2

    out = pl.pallas_call(
        functools.partial(_body, n_pos=n_pos, scale=scale),
        grid=(b // blk,),
        in_specs=[
            pl.BlockSpec(memory_space=pltpu.SMEM),
            pl.BlockSpec((blk, s // 2, 2 * dim), lambda i: (i, 0, 0)),
            pl.BlockSpec((2 * dim, 2 * dim), lambda i: (0, 0)),
        ],
        out_specs=pl.BlockSpec((blk, s // 2, 2 * dim), lambda i: (i, 0, 0)),
        out_shape=jax.ShapeDtypeStruct((b, s // 2, 2 * dim), jnp.float32),
        compiler_params=pltpu.CompilerParams(
            dimension_semantics=("parallel",),
        ),
    )(cb2, x2, rot2)
    return out.reshape(b, s, dim)


# speedup vs baseline: 4.0843x; 4.0843x over previous
import functools

import jax
import jax.numpy as jnp
from jax import lax
from jax.experimental import pallas as pl
from jax.experimental.pallas import tpu as pltpu


def _body(cb_ref, x_ref, rot_ref, o_ref, *, n_pos, scale, bblk):
    rot = rot_ref[...]
    c = [cb_ref[0, n_pos + j] * scale for j in range(n_pos)]
    mids = [(c[j - 1] + c[j]) * 0.5 for j in range(1, n_pos)]

    for b in range(bblk):
        xb = x_ref[b]
        y = jnp.dot(rot, xb, preferred_element_type=jnp.float32)
        a = jnp.abs(y)

        def tree(lo, hi):
            if lo == hi:
                return jnp.full_like(a, c[lo])
            m = (lo + hi) // 2
            return jnp.where(a > mids[m], tree(m + 1, hi), tree(lo, m))

        q = tree(0, n_pos - 1)
        yq = jnp.where(y < 0.0, -q, q)
        o_ref[b] = lax.dot_general(rot, yq, (((0,), (0,)), ((), ())),
                                   preferred_element_type=jnp.float32)


def kernel(x, rotation, codebook):
    b, s, dim = x.shape
    scale = 1.0 / (dim ** 0.5)
    k = codebook.shape[0]
    n_pos = k // 2
    cb2 = codebook.reshape(1, k)

    xt = jnp.transpose(x, (0, 2, 1))

    bblk = 2
    while b % bblk:
        bblk //= 2

    out = pl.pallas_call(
        functools.partial(_body, n_pos=n_pos, scale=scale, bblk=bblk),
        grid=(b // bblk,),
        in_specs=[
            pl.BlockSpec(memory_space=pltpu.SMEM),
            pl.BlockSpec((bblk, dim, s), lambda i: (i, 0, 0)),
            pl.BlockSpec((dim, dim), lambda i: (0, 0)),
        ],
        out_specs=pl.BlockSpec((bblk, dim, s), lambda i: (i, 0, 0)),
        out_shape=jax.ShapeDtypeStruct((b, dim, s), jnp.float32),
        compiler_params=pltpu.CompilerParams(
            dimension_semantics=("parallel",),
        ),
    )(cb2, xt, rotation)
    return jnp.transpose(out, (0, 2, 1))


# speedup vs baseline: 5.3985x; 1.3218x over previous
import functools

import jax
import jax.numpy as jnp
from jax import lax
from jax.experimental import pallas as pl
from jax.experimental.pallas import tpu as pltpu


def _body(cb_ref, x_ref, rot_ref, o_ref, *, n_pos, scale, bblk):
    rot = rot_ref[...]
    c = [cb_ref[0, n_pos + j] * scale for j in range(n_pos)]
    mids = [(c[j - 1] + c[j]) * 0.5 for j in range(1, n_pos)]

    for b in range(bblk):
        xb = x_ref[b]
        y = jnp.dot(rot, xb, preferred_element_type=jnp.float32)
        a = jnp.abs(y)

        def tree(lo, hi):
            if lo == hi:
                return jnp.full_like(a, c[lo])
            m = (lo + hi) // 2
            return jnp.where(a > mids[m], tree(m + 1, hi), tree(lo, m))

        q = tree(0, n_pos - 1)
        yq = lax.bitcast_convert_type(
            lax.bitcast_convert_type(q, jnp.uint32)
            | (lax.bitcast_convert_type(y, jnp.uint32) & jnp.uint32(0x80000000)),
            jnp.float32)
        o_ref[b] = lax.dot_general(rot, yq, (((0,), (0,)), ((), ())),
                                   preferred_element_type=jnp.float32)


def kernel(x, rotation, codebook):
    b, s, dim = x.shape
    scale = 1.0 / (dim ** 0.5)
    k = codebook.shape[0]
    n_pos = k // 2
    cb2 = codebook.reshape(1, k)

    xt = jnp.transpose(x, (0, 2, 1))

    bblk = 8
    while b % bblk:
        bblk //= 2

    out = pl.pallas_call(
        functools.partial(_body, n_pos=n_pos, scale=scale, bblk=bblk),
        grid=(b // bblk,),
        in_specs=[
            pl.BlockSpec(memory_space=pltpu.SMEM),
            pl.BlockSpec((bblk, dim, s), lambda i: (i, 0, 0)),
            pl.BlockSpec((dim, dim), lambda i: (0, 0)),
        ],
        out_specs=pl.BlockSpec((bblk, dim, s), lambda i: (i, 0, 0)),
        out_shape=jax.ShapeDtypeStruct((b, dim, s), jnp.float32),
        compiler_params=pltpu.CompilerParams(
            dimension_semantics=("parallel",),
        ),
    )(cb2, xt, rotation)
    return jnp.transpose(out, (0, 2, 1))


# speedup vs baseline: 5.6398x; 1.0447x over previous
import functools

import jax
import jax.numpy as jnp
from jax import lax
from jax.experimental import pallas as pl
from jax.experimental.pallas import tpu as pltpu

MERGE = 4
BBLK = 8


def _body(cb_ref, x_ref, rot_ref, o_ref, *, n_pos, scale, merge, dim):
    rot = rot_ref[...]
    c = [cb_ref[0, n_pos + j] * scale for j in range(n_pos)]
    mids = [(c[j - 1] + c[j]) * 0.5 for j in range(1, n_pos)]

    bblk = x_ref.shape[0]
    xall = x_ref[...]
    for g in range(bblk // merge):
        xb = xall[g * merge:(g + 1) * merge].reshape(merge * dim, -1)
        y = jnp.dot(rot, xb, preferred_element_type=jnp.float32)
        a = jnp.abs(y)

        def tree(lo, hi):
            if lo == hi:
                return jnp.full_like(a, c[lo])
            m = (lo + hi) // 2
            return jnp.where(a > mids[m], tree(m + 1, hi), tree(lo, m))

        q = tree(0, n_pos - 1)
        yq = lax.bitcast_convert_type(
            lax.bitcast_convert_type(q, jnp.uint32)
            | (lax.bitcast_convert_type(y, jnp.uint32) & jnp.uint32(0x80000000)),
            jnp.float32)
        o = lax.dot_general(rot, yq, (((0,), (0,)), ((), ())),
                            preferred_element_type=jnp.float32)
        o_ref[g * merge:(g + 1) * merge] = o.reshape(merge, dim, -1)


def kernel(x, rotation, codebook):
    b, s, dim = x.shape
    scale = 1.0 / (dim ** 0.5)
    k = codebook.shape[0]
    n_pos = k // 2
    cb2 = codebook.reshape(1, k)

    xt = jnp.transpose(x, (0, 2, 1))

    merge = MERGE
    rot2 = jnp.kron(jnp.eye(merge, dtype=rotation.dtype), rotation)

    bblk = BBLK
    while b % bblk:
        bblk //= 2

    out = pl.pallas_call(
        functools.partial(_body, n_pos=n_pos, scale=scale, merge=merge, dim=dim),
        grid=(b // bblk,),
        in_specs=[
            pl.BlockSpec(memory_space=pltpu.SMEM),
            pl.BlockSpec((bblk, dim, s), lambda i: (i, 0, 0)),
            pl.BlockSpec((merge * dim, merge * dim), lambda i: (0, 0)),
        ],
        out_specs=pl.BlockSpec((bblk, dim, s), lambda i: (i, 0, 0)),
        out_shape=jax.ShapeDtypeStruct((b, dim, s), jnp.float32),
        compiler_params=pltpu.CompilerParams(
            dimension_semantics=("parallel",),
        ),
    )(cb2, xt, rot2)
    return jnp.transpose(out, (0, 2, 1))


# speedup vs baseline: 6.5729x; 1.1655x over previous
import functools

import jax
import jax.numpy as jnp
from jax import lax
from jax.experimental import pallas as pl
from jax.experimental.pallas import tpu as pltpu

MERGE = 4
BBLK = 8


def _body(cb_ref, x_ref, rot_ref, o_ref, rot4_ref, *, n_pos, scale, merge, dim):
    @pl.when(pl.program_id(0) == 0)
    def _build_blockdiag():
        rot4_ref[...] = jnp.zeros_like(rot4_ref)
        rot = rot_ref[...]
        for g in range(merge):
            rot4_ref[pl.ds(g * dim, dim), pl.ds(g * dim, dim)] = rot

    rot4 = rot4_ref[...]
    c = [cb_ref[0, n_pos + j] * scale for j in range(n_pos)]
    mids = [(c[j - 1] + c[j]) * 0.5 for j in range(1, n_pos)]

    bblk = x_ref.shape[0]
    xall = x_ref[...]
    for g in range(bblk // merge):
        xb = xall[g * merge:(g + 1) * merge].reshape(merge * dim, -1)
        y = jnp.dot(rot4, xb, preferred_element_type=jnp.float32)
        a = jnp.abs(y)

        def tree(lo, hi):
            if lo == hi:
                return jnp.full_like(a, c[lo])
            m = (lo + hi) // 2
            return jnp.where(a > mids[m], tree(m + 1, hi), tree(lo, m))

        q = tree(0, n_pos - 1)
        yq = lax.bitcast_convert_type(
            lax.bitcast_convert_type(q, jnp.uint32)
            | (lax.bitcast_convert_type(y, jnp.uint32) & jnp.uint32(0x80000000)),
            jnp.float32)
        o = lax.dot_general(rot4, yq, (((0,), (0,)), ((), ())),
                            preferred_element_type=jnp.float32)
        o_ref[g * merge:(g + 1) * merge] = o.reshape(merge, dim, -1)


def kernel(x, rotation, codebook):
    b, s, dim = x.shape
    scale = 1.0 / (dim ** 0.5)
    k = codebook.shape[0]
    n_pos = k // 2
    cb2 = codebook.reshape(1, k)

    xt = jnp.transpose(x, (0, 2, 1))

    bblk = BBLK
    while b % bblk:
        bblk //= 2
    merge = min(MERGE, bblk)

    out = pl.pallas_call(
        functools.partial(_body, n_pos=n_pos, scale=scale, merge=merge, dim=dim),
        grid=(b // bblk,),
        in_specs=[
            pl.BlockSpec(memory_space=pltpu.SMEM),
            pl.BlockSpec((bblk, dim, s), lambda i: (i, 0, 0)),
            pl.BlockSpec((dim, dim), lambda i: (0, 0)),
        ],
        out_specs=pl.BlockSpec((bblk, dim, s), lambda i: (i, 0, 0)),
        out_shape=jax.ShapeDtypeStruct((b, dim, s), jnp.float32),
        scratch_shapes=[pltpu.VMEM((merge * dim, merge * dim), jnp.float32)],
        compiler_params=pltpu.CompilerParams(
            dimension_semantics=("arbitrary",),
        ),
    )(cb2, xt, rotation)
    return jnp.transpose(out, (0, 2, 1))
